# baseline (speedup 1.0000x reference)
import functools

import jax
import jax.numpy as jnp
from jax import lax
from jax.experimental import pallas as pl
from jax.experimental.pallas import tpu as pltpu
from jax.experimental.pallas import tpu_sc as plsc

_N = 10000
_D = 128
_E = 320000
_LANES = 80
_ROWS_T = 125
_NC = 2
_NS = 16
_NPAD = 10240
_RPS = _NPAD // _NS
_CPS = [80] * 8
_BLK = 400
_GRID = _N // _BLK

_mesh = plsc.VectorSubcoreMesh(core_axis_name="c", subcore_axis_name="s")


def _sc_body(with_cnt, *refs):
    if with_cnt:
        (table, src1d, dst1d, parts_out, cnt_out,
         acc_sh, cnt_sh, srcb0, srcb1, dstb0, dstb1, dstb2, rows0, rows1,
         onesb, cvec,
         semg0, semg1, sems0, sems1, semc0, semc1, semi0, semi1) = refs
        semc = (semc0, semc1)
    else:
        (table, src1d, dst1d, parts_out,
         acc_sh, srcb0, srcb1, dstb0, dstb1, dstb2, rows0, rows1,
         semg0, semg1, sems0, sems1, semi0, semi1) = refs
    srcb = (srcb0, srcb1)
    dstb = (dstb0, dstb1, dstb2)
    rows = (rows0, rows1)
    semg = (semg0, semg1)
    sems = (sems0, sems1)
    semi = (semi0, semi1)

    cid = lax.axis_index("c")
    sid = lax.axis_index("s")
    base = sid * _RPS

    def _zrow(i, c):
        for j in range(_D // 16):
            rows0[i, pl.ds(j * 16, 16)] = jnp.zeros((16,), jnp.float32)
        return c
    lax.fori_loop(0, _LANES, _zrow, 0)
    off = 0
    for w in _CPS:
        pltpu.sync_copy(rows0.at[pl.ds(0, w)], acc_sh.at[pl.ds(base + off, w)])
        off += w
    if with_cnt:
        def _zc(i, c):
            cvec[pl.ds(i * 16, 16)] = jnp.zeros((16,), jnp.float32)
            return c
        lax.fori_loop(0, _RPS // 16, _zc, 0)
        pltpu.sync_copy(cvec, cnt_sh.at[pl.ds(base, _RPS)])
        def _oc(i, c):
            onesb[pl.ds(i * 16, 16)] = jnp.ones((16,), jnp.float32)
            return c
        lax.fori_loop(0, _LANES // 16, _oc, 0)
    plsc.subcore_barrier()

    def _zrow1(i, c):
        for j in range(_D // 16):
            rows1[i, pl.ds(j * 16, 16)] = jnp.zeros((16,), jnp.float32)
        return c
    lax.fori_loop(0, _LANES, _zrow1, 0)
    def _zd2(i, c):
        dstb2[pl.ds(i * 16, 16)] = jnp.zeros((16,), jnp.int32)
        return c
    lax.fori_loop(0, _LANES // 16, _zd2, 0)
    plsc.subcore_barrier()

    ebase = (cid * _NS + sid) * _ROWS_T * _LANES
    _LAST = _ROWS_T - 1

    def _load(j, sb, db, sem):
        off = ebase + jnp.minimum(j, _LAST) * _LANES
        pltpu.async_copy(src1d.at[pl.ds(off, _LANES)], sb, sem)
        pltpu.async_copy(dst1d.at[pl.ds(off, _LANES)], db, sem)

    def _wload(sb, db, sem):
        pltpu.make_async_copy(src1d.at[pl.ds(0, _LANES)], sb, sem).wait()
        pltpu.make_async_copy(dst1d.at[pl.ds(0, _LANES)], db, sem).wait()

    _load(0, srcb0, dstb0, semi0)
    _wload(srcb0, dstb0, semi0)
    pltpu.async_copy(table.at[srcb0], rows0, semg0)
    _load(1, srcb1, dstb1, semi1)
    pltpu.async_copy(rows1, acc_sh.at[dstb2], sems1, add=True)
    if with_cnt:
        pltpu.async_copy(cvec.at[pl.ds(0, _LANES)], cnt_sh.at[dstb2],
                         semc1, add=True)

    def _step(jj, u):
        p, q = u % 2, (u + 1) % 2
        d, dn, dp = u % 3, (u + 1) % 3, (u + 2) % 3
        _wload(srcb[q], dstb[dn], semi[q])
        pltpu.make_async_copy(table.at[srcb[p]], rows[p], semg[p]).wait()
        pltpu.make_async_copy(rows[q], acc_sh.at[dstb[dp]], sems[q]).wait()
        if with_cnt:
            pltpu.make_async_copy(onesb, cnt_sh.at[dstb[dp]], semc[q]).wait()
        pltpu.async_copy(rows[p], acc_sh.at[dstb[d]], sems[p], add=True)
        if with_cnt:
            pltpu.async_copy(onesb, cnt_sh.at[dstb[d]], semc[p], add=True)
        pltpu.async_copy(table.at[srcb[q]], rows[q], semg[q])
        _load(jj + 2, srcb[p], dstb[dp], semi[p])

    def _six(g, c):
        for u in range(6):
            _step(6 * g + u, u)
        return c
    lax.fori_loop(0, _ROWS_T // 6, _six, 0)
    for u in range(_ROWS_T % 6):
        _step((_ROWS_T // 6) * 6 + u, u)

    _p, _q = _LAST % 2, (_LAST + 1) % 2
    _d, _dn = _LAST % 3, (_LAST + 2) % 3
    pltpu.make_async_copy(table.at[srcb[_q]], rows[_q], semg[_q]).wait()
    pltpu.make_async_copy(rows[_p], acc_sh.at[dstb[_d]], sems[_p]).wait()
    if with_cnt:
        pltpu.make_async_copy(onesb, cnt_sh.at[dstb[_d]], semc[_p]).wait()
    _wload(srcb[_p], dstb[_dn], semi[_p])

    plsc.subcore_barrier()

    obase = cid * _NPAD + base
    off = 0
    for w in _CPS:
        pltpu.sync_copy(acc_sh.at[pl.ds(base + off, w)], rows0.at[pl.ds(0, w)])
        pltpu.sync_copy(rows0.at[pl.ds(0, w)],
                        parts_out.at[pl.ds(obase + off, w)])
        off += w
    if with_cnt:
        pltpu.sync_copy(cnt_sh.at[pl.ds(base, _RPS)], cvec)
        pltpu.sync_copy(cvec, cnt_out.at[pl.ds(obase, _RPS)])


def _make_sc(with_cnt):
    out_type = [jax.ShapeDtypeStruct((_NC * _NPAD, _D), jnp.float32)]
    scratch = [
        pltpu.VMEM_SHARED((_NPAD, _D), jnp.float32),
    ]
    if with_cnt:
        out_type.append(jax.ShapeDtypeStruct((_NC * _NPAD,), jnp.float32))
        scratch.append(pltpu.VMEM_SHARED((_NPAD,), jnp.float32))
    scratch += [
        pltpu.VMEM((_LANES,), jnp.int32),
        pltpu.VMEM((_LANES,), jnp.int32),
        pltpu.VMEM((_LANES,), jnp.int32),
        pltpu.VMEM((_LANES,), jnp.int32),
        pltpu.VMEM((_LANES,), jnp.int32),
        pltpu.VMEM((_LANES, _D), jnp.float32),
        pltpu.VMEM((_LANES, _D), jnp.float32),
    ]
    if with_cnt:
        scratch.append(pltpu.VMEM((_LANES,), jnp.float32))
        scratch.append(pltpu.VMEM((_RPS,), jnp.float32))
    scratch += [pltpu.SemaphoreType.DMA] * (8 if with_cnt else 6)
    return pl.kernel(
        functools.partial(_sc_body, with_cnt),
        out_type=tuple(out_type) if with_cnt else out_type[0],
        mesh=_mesh,
        scratch_types=tuple(scratch),
    )


_sc_agg_cnt = _make_sc(True)
_sc_agg = _make_sc(False)


def _l0_body(parts, cnt, x, wl, wr, b, h_out):
    a = parts[0] + parts[1]
    c = cnt[0, 0] + cnt[0, 1]
    inv = (1.0 / jnp.maximum(c, 1.0))[:, None]
    h = (jnp.dot(a * inv, wl[...], preferred_element_type=jnp.float32) + b[...]
         + jnp.dot(x[...], wr[...], preferred_element_type=jnp.float32))
    h_out[...] = jnp.maximum(h, 0.0)


def _l1_body(parts, cnt, h0, wl, wr, b, wg, bg, wo, bo, out, acc):
    i = pl.program_id(0)
    a = parts[0] + parts[1]
    c = cnt[0, 0] + cnt[0, 1]
    inv = (1.0 / jnp.maximum(c, 1.0))[:, None]
    h = (jnp.dot(a * inv, wl[...], preferred_element_type=jnp.float32) + b[...]
         + jnp.dot(h0[...], wr[...], preferred_element_type=jnp.float32))
    h = jnp.maximum(h, 0.0)
    s = jnp.sum(h, axis=0, keepdims=True)

    @pl.when(i == 0)
    def _():
        acc[...] = s

    @pl.when(i > 0)
    def _():
        acc[...] = acc[...] + s

    g = acc[...] * (1.0 / _N)
    z = jnp.maximum(
        jnp.dot(g, wg[...], preferred_element_type=jnp.float32) + bg[...], 0.0)
    out[...] = jnp.dot(z, wo[...], preferred_element_type=jnp.float32) + bo[...]


def _l0_call(parts, cnt, x, wlT, wrT, bl):
    return pl.pallas_call(
        _l0_body,
        grid=(_GRID,),
        in_specs=[
            pl.BlockSpec((_NC, _BLK, _D), lambda i: (0, i, 0)),
            pl.BlockSpec((1, _NC, _BLK), lambda i: (i, 0, 0)),
            pl.BlockSpec((_BLK, _D), lambda i: (i, 0)),
            pl.BlockSpec((_D, _D), lambda i: (0, 0)),
            pl.BlockSpec((_D, _D), lambda i: (0, 0)),
            pl.BlockSpec((1, _D), lambda i: (0, 0)),
        ],
        out_specs=pl.BlockSpec((_BLK, _D), lambda i: (i, 0)),
        out_shape=jax.ShapeDtypeStruct((_N, _D), jnp.float32),
    )(parts, cnt, x, wlT, wrT, bl)


def _l1_call(parts, cnt, h0, wlT, wrT, bl, wgT, bg, woT, bo):
    return pl.pallas_call(
        _l1_body,
        grid=(_GRID,),
        in_specs=[
            pl.BlockSpec((_NC, _BLK, _D), lambda i: (0, i, 0)),
            pl.BlockSpec((1, _NC, _BLK), lambda i: (i, 0, 0)),
            pl.BlockSpec((_BLK, _D), lambda i: (i, 0)),
            pl.BlockSpec((_D, _D), lambda i: (0, 0)),
            pl.BlockSpec((_D, _D), lambda i: (0, 0)),
            pl.BlockSpec((1, _D), lambda i: (0, 0)),
            pl.BlockSpec((_D, _D), lambda i: (0, 0)),
            pl.BlockSpec((1, _D), lambda i: (0, 0)),
            pl.BlockSpec((_D, 16), lambda i: (0, 0)),
            pl.BlockSpec((1, 16), lambda i: (0, 0)),
        ],
        out_specs=pl.BlockSpec((1, 16), lambda i: (0, 0)),
        out_shape=jax.ShapeDtypeStruct((1, 16), jnp.float32),
        scratch_shapes=[pltpu.VMEM((1, _D), jnp.float32)],
    )(parts, cnt, h0, wlT, wrT, bl, wgT, bg, woT, bo)


def kernel(x, edge_index, batch, Wl0, bl0, Wr0, Wl1, bl1, Wr1, Wlin0, blin0, Wout, bout):
    src = edge_index[0]
    dst = edge_index[1]
    parts0, cnt = _sc_agg_cnt(x, src, dst)
    parts0 = parts0.reshape(_NC, _NPAD, _D)
    cnt = (cnt.reshape(_NC, _NPAD)[:, :_N]
           .reshape(_NC, _GRID, _BLK).transpose(1, 0, 2))
    h0 = _l0_call(parts0, cnt, x, Wl0.T, Wr0.T, bl0[None, :])
    parts1 = _sc_agg(h0, src, dst).reshape(_NC, _NPAD, _D)
    out = _l1_call(parts1, cnt, h0, Wl1.T, Wr1.T, bl1[None, :],
                   Wlin0.T, blin0[None, :], Wout.T, bout[None, :])
    return out


# speedup vs baseline: 1.1136x; 1.1136x over previous
import functools

import jax
import jax.numpy as jnp
from jax import lax
from jax.experimental import pallas as pl
from jax.experimental.pallas import tpu as pltpu
from jax.experimental.pallas import tpu_sc as plsc

_N = 10000
_D = 128
_E = 320000
_LANES = 112
_ROWS_T = 90
_EPAD = _LANES * _ROWS_T * 32
_NC = 2
_NS = 16
_NPAD = 10240
_RPS = _NPAD // _NS
_CPS = [80] * 8
_BLK = 400
_GRID = _N // _BLK

_mesh = plsc.VectorSubcoreMesh(core_axis_name="c", subcore_axis_name="s")


def _sc_body(with_cnt, *refs):
    if with_cnt:
        (table, src1d, dst1d, parts_out, cnt_out,
         acc_sh, cnt_sh, srcb0, srcb1, dstb0, dstb1, dstb2, rows0, rows1,
         onesb, cvec,
         semg0, semg1, sems0, sems1, semc0, semc1, semi0, semi1) = refs
        semc = (semc0, semc1)
    else:
        (table, src1d, dst1d, parts_out,
         acc_sh, srcb0, srcb1, dstb0, dstb1, dstb2, rows0, rows1,
         semg0, semg1, sems0, sems1, semi0, semi1) = refs
    srcb = (srcb0, srcb1)
    dstb = (dstb0, dstb1, dstb2)
    rows = (rows0, rows1)
    semg = (semg0, semg1)
    sems = (sems0, sems1)
    semi = (semi0, semi1)

    cid = lax.axis_index("c")
    sid = lax.axis_index("s")
    base = sid * _RPS

    def _zrow(i, c):
        for j in range(_D // 16):
            rows0[i, pl.ds(j * 16, 16)] = jnp.zeros((16,), jnp.float32)
        return c
    lax.fori_loop(0, _LANES, _zrow, 0)
    off = 0
    for w in _CPS:
        pltpu.sync_copy(rows0.at[pl.ds(0, w)], acc_sh.at[pl.ds(base + off, w)])
        off += w
    if with_cnt:
        def _zc(i, c):
            cvec[pl.ds(i * 16, 16)] = jnp.zeros((16,), jnp.float32)
            return c
        lax.fori_loop(0, _RPS // 16, _zc, 0)
        pltpu.sync_copy(cvec, cnt_sh.at[pl.ds(base, _RPS)])
        def _oc(i, c):
            onesb[pl.ds(i * 16, 16)] = jnp.ones((16,), jnp.float32)
            return c
        lax.fori_loop(0, _LANES // 16, _oc, 0)
    plsc.subcore_barrier()

    def _zrow1(i, c):
        for j in range(_D // 16):
            rows1[i, pl.ds(j * 16, 16)] = jnp.zeros((16,), jnp.float32)
        return c
    lax.fori_loop(0, _LANES, _zrow1, 0)
    def _zd2(i, c):
        dstb2[pl.ds(i * 16, 16)] = jnp.zeros((16,), jnp.int32)
        return c
    lax.fori_loop(0, _LANES // 16, _zd2, 0)
    plsc.subcore_barrier()

    ebase = (cid * _NS + sid) * _ROWS_T * _LANES
    _LAST = _ROWS_T - 1

    def _load(j, sb, db, sem):
        off = ebase + jnp.minimum(j, _LAST) * _LANES
        pltpu.async_copy(src1d.at[pl.ds(off, _LANES)], sb, sem)
        pltpu.async_copy(dst1d.at[pl.ds(off, _LANES)], db, sem)

    def _wload(sb, db, sem):
        pltpu.make_async_copy(src1d.at[pl.ds(0, _LANES)], sb, sem).wait()
        pltpu.make_async_copy(dst1d.at[pl.ds(0, _LANES)], db, sem).wait()

    _load(0, srcb0, dstb0, semi0)
    _wload(srcb0, dstb0, semi0)
    pltpu.async_copy(table.at[srcb0], rows0, semg0)
    _load(1, srcb1, dstb1, semi1)
    pltpu.async_copy(rows1, acc_sh.at[dstb2], sems1, add=True)
    if with_cnt:
        pltpu.async_copy(cvec.at[pl.ds(0, _LANES)], cnt_sh.at[dstb2],
                         semc1, add=True)

    def _step(jj, u):
        p, q = u % 2, (u + 1) % 2
        d, dn, dp = u % 3, (u + 1) % 3, (u + 2) % 3
        _wload(srcb[q], dstb[dn], semi[q])
        pltpu.make_async_copy(table.at[srcb[p]], rows[p], semg[p]).wait()
        pltpu.make_async_copy(rows[q], acc_sh.at[dstb[dp]], sems[q]).wait()
        if with_cnt:
            pltpu.make_async_copy(onesb, cnt_sh.at[dstb[dp]], semc[q]).wait()
        pltpu.async_copy(rows[p], acc_sh.at[dstb[d]], sems[p], add=True)
        if with_cnt:
            pltpu.async_copy(onesb, cnt_sh.at[dstb[d]], semc[p], add=True)
        pltpu.async_copy(table.at[srcb[q]], rows[q], semg[q])
        _load(jj + 2, srcb[p], dstb[dp], semi[p])

    def _six(g, c):
        for u in range(6):
            _step(6 * g + u, u)
        return c
    lax.fori_loop(0, _ROWS_T // 6, _six, 0)
    for u in range(_ROWS_T % 6):
        _step((_ROWS_T // 6) * 6 + u, u)

    _p, _q = _LAST % 2, (_LAST + 1) % 2
    _d, _dn = _LAST % 3, (_LAST + 2) % 3
    pltpu.make_async_copy(table.at[srcb[_q]], rows[_q], semg[_q]).wait()
    pltpu.make_async_copy(rows[_p], acc_sh.at[dstb[_d]], sems[_p]).wait()
    if with_cnt:
        pltpu.make_async_copy(onesb, cnt_sh.at[dstb[_d]], semc[_p]).wait()
    _wload(srcb[_p], dstb[_dn], semi[_p])

    plsc.subcore_barrier()

    obase = cid * _NPAD + base
    off = 0
    for w in _CPS:
        pltpu.sync_copy(acc_sh.at[pl.ds(base + off, w)], rows0.at[pl.ds(0, w)])
        pltpu.sync_copy(rows0.at[pl.ds(0, w)],
                        parts_out.at[pl.ds(obase + off, w)])
        off += w
    if with_cnt:
        pltpu.sync_copy(cnt_sh.at[pl.ds(base, _RPS)], cvec)
        pltpu.sync_copy(cvec, cnt_out.at[pl.ds(obase, _RPS)])


def _make_sc(with_cnt):
    out_type = [jax.ShapeDtypeStruct((_NC * _NPAD, _D), jnp.float32)]
    scratch = [
        pltpu.VMEM_SHARED((_NPAD, _D), jnp.float32),
    ]
    if with_cnt:
        out_type.append(jax.ShapeDtypeStruct((_NC * _NPAD,), jnp.float32))
        scratch.append(pltpu.VMEM_SHARED((_NPAD,), jnp.float32))
    scratch += [
        pltpu.VMEM((_LANES,), jnp.int32),
        pltpu.VMEM((_LANES,), jnp.int32),
        pltpu.VMEM((_LANES,), jnp.int32),
        pltpu.VMEM((_LANES,), jnp.int32),
        pltpu.VMEM((_LANES,), jnp.int32),
        pltpu.VMEM((_LANES, _D), jnp.float32),
        pltpu.VMEM((_LANES, _D), jnp.float32),
    ]
    if with_cnt:
        scratch.append(pltpu.VMEM((_LANES,), jnp.float32))
        scratch.append(pltpu.VMEM((_RPS,), jnp.float32))
    scratch += [pltpu.SemaphoreType.DMA] * (8 if with_cnt else 6)
    return pl.kernel(
        functools.partial(_sc_body, with_cnt),
        out_type=tuple(out_type) if with_cnt else out_type[0],
        mesh=_mesh,
        scratch_types=tuple(scratch),
    )


_sc_agg_cnt = _make_sc(True)
_sc_agg = _make_sc(False)


def _l0_body(parts, cnt, x, wl, wr, b, h_out):
    a = parts[0] + parts[1]
    c = cnt[0, 0] + cnt[0, 1]
    inv = (1.0 / jnp.maximum(c, 1.0))[:, None]
    h = (jnp.dot(a * inv, wl[...], preferred_element_type=jnp.float32) + b[...]
         + jnp.dot(x[...], wr[...], preferred_element_type=jnp.float32))
    h_out[...] = jnp.maximum(h, 0.0)


def _l1_body(parts, cnt, h0, wl, wr, b, wg, bg, wo, bo, out, acc):
    i = pl.program_id(0)
    a = parts[0] + parts[1]
    c = cnt[0, 0] + cnt[0, 1]
    inv = (1.0 / jnp.maximum(c, 1.0))[:, None]
    h = (jnp.dot(a * inv, wl[...], preferred_element_type=jnp.float32) + b[...]
         + jnp.dot(h0[...], wr[...], preferred_element_type=jnp.float32))
    h = jnp.maximum(h, 0.0)
    s = jnp.sum(h, axis=0, keepdims=True)

    @pl.when(i == 0)
    def _():
        acc[...] = s

    @pl.when(i > 0)
    def _():
        acc[...] = acc[...] + s

    g = acc[...] * (1.0 / _N)
    z = jnp.maximum(
        jnp.dot(g, wg[...], preferred_element_type=jnp.float32) + bg[...], 0.0)
    out[...] = jnp.dot(z, wo[...], preferred_element_type=jnp.float32) + bo[...]


def _l0_call(parts, cnt, x, wlT, wrT, bl):
    return pl.pallas_call(
        _l0_body,
        grid=(_GRID,),
        in_specs=[
            pl.BlockSpec((_NC, _BLK, _D), lambda i: (0, i, 0)),
            pl.BlockSpec((1, _NC, _BLK), lambda i: (i, 0, 0)),
            pl.BlockSpec((_BLK, _D), lambda i: (i, 0)),
            pl.BlockSpec((_D, _D), lambda i: (0, 0)),
            pl.BlockSpec((_D, _D), lambda i: (0, 0)),
            pl.BlockSpec((1, _D), lambda i: (0, 0)),
        ],
        out_specs=pl.BlockSpec((_BLK, _D), lambda i: (i, 0)),
        out_shape=jax.ShapeDtypeStruct((_N, _D), jnp.float32),
    )(parts, cnt, x, wlT, wrT, bl)


def _l1_call(parts, cnt, h0, wlT, wrT, bl, wgT, bg, woT, bo):
    return pl.pallas_call(
        _l1_body,
        grid=(_GRID,),
        in_specs=[
            pl.BlockSpec((_NC, _BLK, _D), lambda i: (0, i, 0)),
            pl.BlockSpec((1, _NC, _BLK), lambda i: (i, 0, 0)),
            pl.BlockSpec((_BLK, _D), lambda i: (i, 0)),
            pl.BlockSpec((_D, _D), lambda i: (0, 0)),
            pl.BlockSpec((_D, _D), lambda i: (0, 0)),
            pl.BlockSpec((1, _D), lambda i: (0, 0)),
            pl.BlockSpec((_D, _D), lambda i: (0, 0)),
            pl.BlockSpec((1, _D), lambda i: (0, 0)),
            pl.BlockSpec((_D, 16), lambda i: (0, 0)),
            pl.BlockSpec((1, 16), lambda i: (0, 0)),
        ],
        out_specs=pl.BlockSpec((1, 16), lambda i: (0, 0)),
        out_shape=jax.ShapeDtypeStruct((1, 16), jnp.float32),
        scratch_shapes=[pltpu.VMEM((1, _D), jnp.float32)],
    )(parts, cnt, h0, wlT, wrT, bl, wgT, bg, woT, bo)


def kernel(x, edge_index, batch, Wl0, bl0, Wr0, Wl1, bl1, Wr1, Wlin0, blin0, Wout, bout):
    pad_e = _EPAD - _E
    ar = jnp.arange(pad_e, dtype=jnp.int32)
    src = jnp.concatenate([edge_index[0], ar % _N])
    dst = jnp.concatenate([edge_index[1], _N + ar % (_NPAD - _N)])
    parts0, cnt = _sc_agg_cnt(x, src, dst)
    parts0 = parts0.reshape(_NC, _NPAD, _D)
    cnt = (cnt.reshape(_NC, _NPAD)[:, :_N]
           .reshape(_NC, _GRID, _BLK).transpose(1, 0, 2))
    h0 = _l0_call(parts0, cnt, x, Wl0.T, Wr0.T, bl0[None, :])
    parts1 = _sc_agg(h0, src, dst).reshape(_NC, _NPAD, _D)
    out = _l1_call(parts1, cnt, h0, Wl1.T, Wr1.T, bl1[None, :],
                   Wlin0.T, blin0[None, :], Wout.T, bout[None, :])
    return out
